# Initial kernel scaffold; baseline (speedup 1.0000x reference)
import jax
import jax.numpy as jnp
from jax.experimental import pallas as pl


def kernel(pos, edge_index):
    raise NotImplementedError("write your pallas kernel here")


# speedup vs baseline: 12.9283x; 12.9283x over previous
import jax
import jax.numpy as jnp
from jax import lax
from jax.experimental import pallas as pl
from jax.experimental.pallas import tpu as pltpu
from jax.experimental.pallas import tpu_sc as plsc

_N_NODES = 100000
_N_EDGES = 3200000
_OUT_DIM = 16
_CUTOFF = 5.0
_NW = 32
_EPW = _N_EDGES // _NW
_CHUNK = 2000
_NCH = _EPW // _CHUNK
_GROUPS = _CHUNK // 16
_WIDTH = _CUTOFF / (_OUT_DIM - 1)
_NEG_I2W2 = -1.0 / (2.0 * _WIDTH * _WIDTH)
_CENTERS = [_CUTOFF * k / (_OUT_DIM - 1) for k in range(_OUT_DIM)]


def _sc_body(px_hbm, py_hbm, pz_hbm, src_hbm, dst_hbm, out_hbm,
             shx, shy, shz, src_idx, dst_idx,
             xs, ys, zs, xd, yd, zd, emb, sem_s, sem_d):
    sid = lax.axis_index("s")
    wid = sid * 2 + lax.axis_index("c")
    iota = lax.iota(jnp.int32, 16)

    @pl.when(sid == 0)
    def _stage_planes():
        pltpu.sync_copy(px_hbm, shx)
        pltpu.sync_copy(py_hbm, shy)
        pltpu.sync_copy(pz_hbm, shz)

    plsc.subcore_barrier()

    def chunk_body(ci, carry):
        ebase = wid * _EPW + ci * _CHUNK
        pltpu.sync_copy(src_hbm.at[pl.ds(ebase, _CHUNK)], src_idx)
        pltpu.sync_copy(dst_hbm.at[pl.ds(ebase, _CHUNK)], dst_idx)
        cs = [pltpu.async_copy(sh.at[src_idx], v, sem_s)
              for sh, v in ((shx, xs), (shy, ys), (shz, zs))]
        cd = [pltpu.async_copy(sh.at[dst_idx], v, sem_d)
              for sh, v in ((shx, xd), (shy, yd), (shz, zd))]
        for c in cs + cd:
            c.wait()

        def group_body(gi, inner):
            o = pl.ds(gi * 16, 16)
            dx = xs[o] - xd[o]
            dy = ys[o] - yd[o]
            dz = zs[o] - zd[o]
            s = dx * dx + dy * dy + dz * dz
            bits = plsc.bitcast(s, jnp.int32)
            bits = 0x5F3759DF - lax.shift_right_arithmetic(bits, 1)
            y = plsc.bitcast(bits, jnp.float32)
            for _ in range(3):
                y = y * (1.5 - 0.5 * s * y * y)
            r = jnp.where(s > 0.0, s * y, 0.0)
            e16 = (gi * 16 + iota) * _OUT_DIM
            for k in range(_OUT_DIM):
                t = r - _CENTERS[k]
                v = jnp.exp(t * t * _NEG_I2W2)
                plsc.store_scatter(emb, [e16 + k], v)
            return inner

        lax.fori_loop(0, _GROUPS, group_body, 0)
        pltpu.sync_copy(emb, out_hbm.at[pl.ds(ebase * _OUT_DIM,
                                              _CHUNK * _OUT_DIM)])
        return carry

    lax.fori_loop(0, _NCH, chunk_body, 0)


@jax.jit
def _radial(px, py, pz, src, dst):
    f = pl.kernel(
        _sc_body,
        out_type=jax.ShapeDtypeStruct((_N_EDGES * _OUT_DIM,), jnp.float32),
        mesh=plsc.VectorSubcoreMesh(core_axis_name="c", subcore_axis_name="s"),
        scratch_types=[
            pltpu.VMEM_SHARED((_N_NODES,), jnp.float32),
            pltpu.VMEM_SHARED((_N_NODES,), jnp.float32),
            pltpu.VMEM_SHARED((_N_NODES,), jnp.float32),
            pltpu.VMEM((_CHUNK,), jnp.int32),
            pltpu.VMEM((_CHUNK,), jnp.int32),
            pltpu.VMEM((_CHUNK,), jnp.float32),
            pltpu.VMEM((_CHUNK,), jnp.float32),
            pltpu.VMEM((_CHUNK,), jnp.float32),
            pltpu.VMEM((_CHUNK,), jnp.float32),
            pltpu.VMEM((_CHUNK,), jnp.float32),
            pltpu.VMEM((_CHUNK,), jnp.float32),
            pltpu.VMEM((_CHUNK * _OUT_DIM,), jnp.float32),
            pltpu.SemaphoreType.DMA,
            pltpu.SemaphoreType.DMA,
        ],
        compiler_params=pltpu.CompilerParams(
            use_tc_tiling_on_sc=False, needs_layout_passes=False),
    )
    return f(px, py, pz, src, dst)


def kernel(pos, edge_index):
    px, py, pz = pos[:, 0], pos[:, 1], pos[:, 2]
    out = _radial(px, py, pz, edge_index[0], edge_index[1])
    return out.reshape(_N_EDGES, _OUT_DIM)
